# baseline (speedup 1.0000x reference)
import functools
import jax
import jax.numpy as jnp
from jax import lax
from jax.experimental import pallas as pl
from jax.experimental.pallas import tpu as pltpu
from jax.experimental.pallas import tpu_sc as plsc

_SCALE = 8.0
_CB = 8
_S1 = 128
_NBUF = 4


def _scale_body(t_ref, o_ref):
    o_ref[...] = t_ref[...] * _SCALE


def _scale_table(table):
    v, d = table.shape
    br = 1024
    grid = (v + br - 1) // br
    return pl.pallas_call(
        _scale_body,
        out_shape=jax.ShapeDtypeStruct((v, d), table.dtype),
        grid=(grid,),
        in_specs=[pl.BlockSpec((br, d), lambda i: (i, 0))],
        out_specs=pl.BlockSpec((br, d), lambda i: (i, 0)),
    )(table)


@functools.cache
def _make_gather(v, d, bt, s):
    info = plsc.get_sparse_core_info()
    nw = info.num_cores * info.num_subcores
    nc = info.num_cores
    bat_per_w = bt // nw
    n_bat = bat_per_w
    n_chunks = bat_per_w // _CB
    assert bat_per_w % _CB == 0
    s2 = s - _S1
    mesh = plsc.VectorSubcoreMesh(core_axis_name="c", subcore_axis_name="s")

    scratch = (
        [pltpu.VMEM((3, _CB, s), jnp.int32)]
        + [pltpu.VMEM((s, d), jnp.float32) for _ in range(_NBUF)]
        + [pltpu.SemaphoreType.DMA for _ in range(_NBUF)]
        + [pltpu.SemaphoreType.DMA for _ in range(_NBUF)]
        + [pltpu.SemaphoreType.DMA((3,))]
    )

    @functools.partial(
        pl.kernel,
        mesh=mesh,
        out_type=jax.ShapeDtypeStruct((bt, s, d), jnp.float32),
        scratch_types=scratch,
        compiler_params=pltpu.CompilerParams(use_tc_tiling_on_sc=False),
    )
    def gather_kernel(table_hbm, idx_hbm, out_hbm, idx_v, *bufs_and_sems):
        rows = bufs_and_sems[0:_NBUF]
        sem_g = bufs_and_sems[_NBUF:2 * _NBUF]
        sem_w = bufs_and_sems[2 * _NBUF:3 * _NBUF]
        sem_i = bufs_and_sems[3 * _NBUF]
        wid = lax.axis_index("s") * nc + lax.axis_index("c")
        bat0 = wid * bat_per_w

        def idx_chunk_copy(c, ib):
            return pltpu.make_async_copy(
                idx_hbm.at[pl.ds(bat0 + c * _CB, _CB)],
                idx_v.at[ib],
                sem_i.at[ib],
            )

        def gather_copies(ci, r, p):
            return [
                pltpu.make_async_copy(
                    table_hbm.at[idx_v.at[ci, r, pl.ds(0, _S1)]],
                    rows[p].at[pl.ds(0, _S1)],
                    sem_g[p],
                ),
                pltpu.make_async_copy(
                    table_hbm.at[idx_v.at[ci, r, pl.ds(_S1, s2)]],
                    rows[p].at[pl.ds(_S1, s2)],
                    sem_g[p],
                ),
            ]

        def fire_gathers(ci, r, p):
            for cp in gather_copies(ci, r, p):
                cp.start()

        def drain_gathers(ci, r, p):
            for cp in gather_copies(ci, r, p):
                cp.wait()

        def write_copy(g, p):
            return pltpu.make_async_copy(
                rows[p],
                out_hbm.at[bat0 + g],
                sem_w[p],
            )

        idx_chunk_copy(0, 0).start()
        idx_chunk_copy(0, 0).wait()
        idx_chunk_copy(1, 1).start()
        fire_gathers(0, 0, 0)
        fire_gathers(0, 1, 1)

        def chunk_fn(c, carry):
            ci = lax.rem(c, 3)
            ci1 = lax.rem(c + 1, 3)

            @pl.when(c + 2 <= n_chunks - 1)
            def _():
                idx_chunk_copy(c + 2, lax.rem(c + 2, 3)).start()

            for bb in range(_CB):
                g = c * _CB + bb
                p = bb % _NBUF
                p2 = (bb + 2) % _NBUF
                drain_gathers(ci, bb, p)
                write_copy(g, p).start()
                @pl.when(g >= 2)
                def _(p2=p2, g=g):
                    write_copy(g - 2, p2).wait()
                if bb < _CB - 2:
                    fire_gathers(ci, bb + 2, p2)
                else:
                    @pl.when(c + 1 <= n_chunks - 1)
                    def _(ci1=ci1, bb=bb, p2=p2):
                        if bb == _CB - 2:
                            idx_chunk_copy(c + 1, ci1).wait()
                        fire_gathers(ci1, bb + 2 - _CB, p2)
            return carry

        lax.fori_loop(0, n_chunks, chunk_fn, 0)

        write_copy(n_bat - 2, (n_bat - 2) % _NBUF).wait()
        write_copy(n_bat - 1, (n_bat - 1) % _NBUF).wait()

    return gather_kernel


def kernel(x, table):
    v, d = table.shape
    bt, s = x.shape
    scaled = _scale_table(table)
    return _make_gather(v, d, bt, s)(scaled, x)


# speedup vs baseline: 1.3301x; 1.3301x over previous
import functools
import jax
import jax.numpy as jnp
from jax import lax
from jax.experimental import pallas as pl
from jax.experimental.pallas import tpu as pltpu
from jax.experimental.pallas import tpu_sc as plsc

_SCALE = 8.0
_IDXW = 128
_NBUF = 4
_NCOMP = 2
_G = 128
_PADW = 128


def _scale_pad_body(t_ref, o_ref):
    t = t_ref[...]
    o_ref[...] = jnp.concatenate(
        [t * _SCALE, jnp.zeros_like(t)], axis=1)


def _scale_pad_table(table):
    v, d = table.shape
    br = 1024
    grid = (v + br - 1) // br
    return pl.pallas_call(
        _scale_pad_body,
        out_shape=jax.ShapeDtypeStruct((v, 2 * d), table.dtype),
        grid=(grid,),
        in_specs=[pl.BlockSpec((br, d), lambda i: (i, 0))],
        out_specs=pl.BlockSpec((br, 2 * d), lambda i: (i, 0)),
    )(table)


@functools.cache
def _make_gather(v, d, b):
    info = plsc.get_sparse_core_info()
    nw = info.num_cores * info.num_subcores
    nc = info.num_cores
    b_per_w = b // nw
    rows_per_body = _NBUF * _G
    n_bodies = b_per_w // rows_per_body
    assert b_per_w % rows_per_body == 0
    idx_rows_per_body = _NBUF
    mesh = plsc.VectorSubcoreMesh(core_axis_name="c", subcore_axis_name="s")

    scratch = (
        [pltpu.VMEM((3, idx_rows_per_body, _IDXW), jnp.int32)]
        + [pltpu.VMEM((_G, _PADW), jnp.float32) for _ in range(_NBUF)]
        + [pltpu.VMEM((_G, d), jnp.float32) for _ in range(_NCOMP)]
        + [pltpu.SemaphoreType.DMA for _ in range(_NBUF)]
        + [pltpu.SemaphoreType.DMA for _ in range(_NCOMP)]
        + [pltpu.SemaphoreType.DMA((3,))]
    )

    @functools.partial(
        pl.kernel,
        mesh=mesh,
        out_type=jax.ShapeDtypeStruct((b, d), jnp.float32),
        scratch_types=scratch,
    )
    def gather_kernel(table_hbm, idx_hbm, out_hbm, idx_v, *bufs_and_sems):
        rows = bufs_and_sems[:_NBUF]
        comp = bufs_and_sems[_NBUF:_NBUF + _NCOMP]
        sem_g = bufs_and_sems[_NBUF + _NCOMP:2 * _NBUF + _NCOMP]
        sem_w = bufs_and_sems[2 * _NBUF + _NCOMP:2 * _NBUF + 2 * _NCOMP]
        sem_i = bufs_and_sems[2 * _NBUF + 2 * _NCOMP]
        wid = lax.axis_index("s") * nc + lax.axis_index("c")
        idx_row0 = wid * (b_per_w // _IDXW)
        out_row0 = wid * b_per_w

        def idx_chunk_copy(c, ib):
            return pltpu.make_async_copy(
                idx_hbm.at[pl.ds(idx_row0 + c * idx_rows_per_body,
                                 idx_rows_per_body)],
                idx_v.at[ib],
                sem_i.at[ib],
            )

        def write_copy(group, cc):
            return pltpu.make_async_copy(
                comp[cc],
                out_hbm.at[pl.ds(out_row0 + group * _G, _G)],
                sem_w[cc],
            )

        def repack(src, dst):
            @plsc.parallel_loop(0, _G, unroll=8)
            def _(r):
                for q in range(d // 16):
                    dst[r, pl.ds(q * 16, 16)] = src[r, pl.ds(q * 16, 16)]

        idx_chunk_copy(0, 0).start()
        idx_chunk_copy(0, 0).wait()
        idx_chunk_copy(1, 1).start()

        def body_fn(c, carry):
            ci = lax.rem(c, 3)
            @pl.when(c >= 1)
            def _():
                idx_chunk_copy(c, ci).wait()

            @pl.when(c + 2 <= n_bodies - 1)
            def _():
                idx_chunk_copy(c + 2, lax.rem(c + 2, 3)).start()

            gathers = [
                pltpu.async_copy(
                    table_hbm.at[idx_v.at[ci, bb]],
                    rows[bb],
                    sem_g[bb],
                )
                for bb in range(_NBUF)
            ]

            for bb in range(_NBUF):
                gathers[bb].wait()
                cc = bb % _NCOMP
                if bb >= _NCOMP:
                    write_copy(c * _NBUF + bb - _NCOMP, cc).wait()
                else:
                    @pl.when(c >= 1)
                    def _(cc=cc, bb=bb):
                        write_copy((c - 1) * _NBUF + bb + _NCOMP, cc).wait()
                repack(rows[bb], comp[cc])
                write_copy(c * _NBUF + bb, cc).start()
            return carry

        lax.fori_loop(0, n_bodies, body_fn, 0)

        for bb in range(_NBUF - _NCOMP, _NBUF):
            write_copy((n_bodies - 1) * _NBUF + bb, bb % _NCOMP).wait()

    return gather_kernel


def kernel(x, table):
    v, d = table.shape
    bt, s = x.shape
    b = bt * s
    scaled = _scale_pad_table(table)
    idx2d = x.reshape(b // _IDXW, _IDXW)
    out = _make_gather(v, d, b)(scaled, idx2d)
    return out.reshape(bt, s, d)
